# Initial kernel scaffold; baseline (speedup 1.0000x reference)
import jax
import jax.numpy as jnp
from jax.experimental import pallas as pl


def kernel(x, edge_index, batch, w1_0, b1_0, w2_0, b2_0, gamma_0, beta_0, w1_1, b1_1, w2_1, b2_1, gamma_1, beta_1, w1_2, b1_2, w2_2, b2_2, gamma_2, beta_2, w_out, b_out):
    raise NotImplementedError("write your pallas kernel here")


# speedup vs baseline: 4.6722x; 4.6722x over previous
import functools

import jax
import jax.numpy as jnp
from jax import lax
from jax.experimental import pallas as pl
from jax.experimental.pallas import tpu as pltpu
from jax.experimental.pallas import tpu_sc as plsc

N = 10000
E = 320000
D = 128
G = 64
BN_EPS = 1e-5

NC = 2
NS = 16
NW = NC * NS

EDGES_PER_TILE = E // NW
CHUNK = 80
CHUNKS = EDGES_PER_TILE // CHUNK
ROWS_PER_TILE = 624
TAIL_ROW0 = ROWS_PER_TILE * NS
TAIL_ROWS = N - TAIL_ROW0


def _sc_aggregate(h, src, dst, zeros):
    mesh = plsc.VectorSubcoreMesh(core_axis_name="c", subcore_axis_name="s")

    @functools.partial(
        pl.kernel,
        mesh=mesh,
        out_type=jax.ShapeDtypeStruct((NC, N, D), jnp.float32),
        scratch_types=[
            pltpu.VMEM((CHUNK,), jnp.int32),
            pltpu.VMEM((CHUNK,), jnp.int32),
            pltpu.VMEM((CHUNK, D), jnp.float32),
            pltpu.VMEM_SHARED((N, D), jnp.float32),
            pltpu.SemaphoreType.DMA,
        ],
    )
    def agg(h_hbm, src_hbm, dst_hbm, zero_hbm, out_hbm,
            src_v, dst_v, rows_v, acc, sem):
        c = lax.axis_index("c")
        s = lax.axis_index("s")
        wid = s * NC + c
        row0 = s * ROWS_PER_TILE

        @pl.when(c == 0)
        def _():
            pltpu.sync_copy(h_hbm.at[pl.ds(row0, ROWS_PER_TILE)],
                            acc.at[pl.ds(row0, ROWS_PER_TILE)])

            @pl.when(s == NS - 1)
            def _():
                pltpu.sync_copy(h_hbm.at[pl.ds(TAIL_ROW0, TAIL_ROWS)],
                                acc.at[pl.ds(TAIL_ROW0, TAIL_ROWS)])

        @pl.when(c != 0)
        def _():
            pltpu.sync_copy(zero_hbm.at[pl.ds(row0, ROWS_PER_TILE)],
                            acc.at[pl.ds(row0, ROWS_PER_TILE)])

            @pl.when(s == NS - 1)
            def _():
                pltpu.sync_copy(zero_hbm.at[pl.ds(TAIL_ROW0, TAIL_ROWS)],
                                acc.at[pl.ds(TAIL_ROW0, TAIL_ROWS)])

        plsc.subcore_barrier()

        base0 = wid * EDGES_PER_TILE

        def body(j, carry):
            base = base0 + j * CHUNK
            pltpu.sync_copy(src_hbm.at[pl.ds(base, CHUNK)], src_v)
            pltpu.sync_copy(dst_hbm.at[pl.ds(base, CHUNK)], dst_v)
            pltpu.async_copy(h_hbm.at[src_v], rows_v, sem).wait()
            pltpu.sync_copy(rows_v, acc.at[dst_v], add=True)
            return carry

        lax.fori_loop(0, CHUNKS, body, 0)
        plsc.subcore_barrier()
        pltpu.sync_copy(acc.at[pl.ds(row0, ROWS_PER_TILE)],
                        out_hbm.at[c, pl.ds(row0, ROWS_PER_TILE)])

        @pl.when(s == NS - 1)
        def _():
            pltpu.sync_copy(acc.at[pl.ds(TAIL_ROW0, TAIL_ROWS)],
                            out_hbm.at[c, pl.ds(TAIL_ROW0, TAIL_ROWS)])

    return agg(h, src, dst, zeros)


def _mlp_bn(p_ref, w1_ref, b1_ref, w2_ref, b2_ref, g_ref, bt_ref):
    z = p_ref[0] + p_ref[1]
    a = jnp.maximum(
        jnp.dot(z, w1_ref[...], preferred_element_type=jnp.float32)
        + b1_ref[...], 0.0)
    y = (jnp.dot(a, w2_ref[...], preferred_element_type=jnp.float32)
         + b2_ref[...])
    mean = jnp.mean(y, axis=0, keepdims=True)
    var = jnp.mean((y - mean) ** 2, axis=0, keepdims=True)
    hn = (y - mean) * lax.rsqrt(var + BN_EPS) * g_ref[...] + bt_ref[...]
    return jnp.maximum(hn, 0.0)


def _tc_layer(p, w1, b1, w2, b2, gamma, beta):
    def body(p_ref, w1_ref, b1_ref, w2_ref, b2_ref, g_ref, bt_ref, o_ref):
        o_ref[...] = _mlp_bn(p_ref, w1_ref, b1_ref, w2_ref, b2_ref,
                             g_ref, bt_ref)

    return pl.pallas_call(
        body,
        out_shape=jax.ShapeDtypeStruct((N, D), jnp.float32),
    )(p, w1, b1.reshape(1, D), w2, b2.reshape(1, D),
      gamma.reshape(1, D), beta.reshape(1, D))


def _tc_final(p, w1, b1, w2, b2, gamma, beta, batch2d, w_out, b_out):
    def body(p_ref, w1_ref, b1_ref, w2_ref, b2_ref, g_ref, bt_ref,
             batch_ref, wo_ref, bo_ref, o_ref):
        h = _mlp_bn(p_ref, w1_ref, b1_ref, w2_ref, b2_ref, g_ref, bt_ref)
        ids = batch_ref[...]
        onehot = (ids == lax.broadcasted_iota(jnp.int32, (1, G), 1)
                  ).astype(jnp.float32)
        counts = jnp.sum(onehot, axis=0, keepdims=True)
        inv = 1.0 / jnp.maximum(counts, 1.0)
        pooled = lax.dot_general(onehot * inv, h, (((0,), (0,)), ((), ())),
                                 preferred_element_type=jnp.float32)
        o_ref[...] = (jnp.dot(pooled, wo_ref[...],
                              preferred_element_type=jnp.float32)
                      + bo_ref[...])

    return pl.pallas_call(
        body,
        out_shape=jax.ShapeDtypeStruct((G, D), jnp.float32),
    )(p, w1, b1.reshape(1, D), w2, b2.reshape(1, D),
      gamma.reshape(1, D), beta.reshape(1, D),
      batch2d, w_out, b_out.reshape(1, D))


def kernel(x, edge_index, batch,
           w1_0, b1_0, w2_0, b2_0, gamma_0, beta_0,
           w1_1, b1_1, w2_1, b2_1, gamma_1, beta_1,
           w1_2, b1_2, w2_2, b2_2, gamma_2, beta_2,
           w_out, b_out):
    src = edge_index[0]
    dst = edge_index[1]
    zeros = jnp.zeros((N, D), jnp.float32)
    batch2d = batch.reshape(N, 1)

    p = _sc_aggregate(x, src, dst, zeros)
    h = _tc_layer(p, w1_0, b1_0, w2_0, b2_0, gamma_0, beta_0)
    p = _sc_aggregate(h, src, dst, zeros)
    h = _tc_layer(p, w1_1, b1_1, w2_1, b2_1, gamma_1, beta_1)
    p = _sc_aggregate(h, src, dst, zeros)
    return _tc_final(p, w1_2, b1_2, w2_2, b2_2, gamma_2, beta_2,
                     batch2d, w_out, b_out)


# speedup vs baseline: 9.8958x; 2.1180x over previous
import functools

import jax
import jax.numpy as jnp
from jax import lax
from jax.experimental import pallas as pl
from jax.experimental.pallas import tpu as pltpu
from jax.experimental.pallas import tpu_sc as plsc

N = 10000
E = 320000
D = 128
H = D // 2
G = 64
BN_EPS = 1e-5

NC = 2
NS = 16
NW = NC * NS

EDGES_PER_TILE = E // NS
CHUNK = 80
CHUNKS = EDGES_PER_TILE // CHUNK
K = 5
PHASES = 2
PCHUNKS = CHUNKS // PHASES
PGROUPS = PCHUNKS // K
PNITER = (PGROUPS - 1) // 2
ROWS_PER_TILE = 624
TAIL_ROW0 = ROWS_PER_TILE * NS
TAIL_ROWS = N - TAIL_ROW0


def _sc_aggregate(h2, src, dst):
    mesh = plsc.VectorSubcoreMesh(core_axis_name="c", subcore_axis_name="s")

    @functools.partial(
        pl.kernel,
        mesh=mesh,
        compiler_params=pltpu.CompilerParams(use_tc_tiling_on_sc=False),
        out_type=jax.ShapeDtypeStruct((NC, N, H), jnp.float32),
        scratch_types=[
            pltpu.VMEM((PCHUNKS, CHUNK), jnp.int32),
            pltpu.VMEM((PCHUNKS, CHUNK), jnp.int32),
            pltpu.VMEM((2 * K, CHUNK, H), jnp.float32),
            pltpu.VMEM_SHARED((N, H), jnp.float32),
            pltpu.SemaphoreType.DMA,
            pltpu.SemaphoreType.DMA,
            pltpu.SemaphoreType.DMA,
            pltpu.SemaphoreType.DMA,
        ],
    )
    def agg(h_hbm, src_hbm, dst_hbm, out_hbm,
            src_v, dst_v, bufs, acc, gsem_a, gsem_b, ssem_a, ssem_b):
        c = lax.axis_index("c")
        s = lax.axis_index("s")
        row0 = s * ROWS_PER_TILE
        half = h_hbm.at[c]

        pltpu.sync_copy(half.at[pl.ds(row0, ROWS_PER_TILE)],
                        acc.at[pl.ds(row0, ROWS_PER_TILE)])

        @pl.when(s == NS - 1)
        def _():
            pltpu.sync_copy(half.at[pl.ds(TAIL_ROW0, TAIL_ROWS)],
                            acc.at[pl.ds(TAIL_ROW0, TAIL_ROWS)])

        plsc.subcore_barrier()

        def issue_gathers(jbase, hb, sem):
            for b in range(K):
                pltpu.async_copy(half.at[src_v.at[jbase + b]],
                                 bufs.at[hb * K + b], sem)

        def drain_gathers(hb, sem):
            for b in range(K):
                pltpu.make_async_copy(half.at[pl.ds(0, CHUNK)],
                                      bufs.at[hb * K + b], sem).wait()

        def issue_scatters(jbase, hb, sem):
            for b in range(K):
                pltpu.async_copy(bufs.at[hb * K + b],
                                 acc.at[dst_v.at[jbase + b]], sem, add=True)

        def drain_scatters(sem):
            for _ in range(K):
                pltpu.make_async_copy(half.at[pl.ds(0, CHUNK)],
                                      acc.at[pl.ds(0, CHUNK)], sem).wait()

        for phase in range(PHASES):
            pltpu.sync_copy(
                src_hbm.at[s, pl.ds(phase * PCHUNKS, PCHUNKS)], src_v)
            pltpu.sync_copy(
                dst_hbm.at[s, pl.ds(phase * PCHUNKS, PCHUNKS)], dst_v)
            issue_gathers(0, 0, gsem_a)

            def sync_scatters(jbase, hb):
                for b in range(K):
                    pltpu.async_copy(bufs.at[hb * K + b],
                                     acc.at[dst_v.at[jbase + b]],
                                     ssem_a, add=True).wait()

            def body(i, carry):
                ja = i * 2 * K
                drain_gathers(0, gsem_a)
                issue_gathers(ja + K, 1, gsem_b)
                sync_scatters(ja, 0)
                drain_gathers(1, gsem_b)
                issue_gathers(ja + 2 * K, 0, gsem_a)
                sync_scatters(ja + K, 1)
                return carry

            lax.fori_loop(0, PNITER, body, 0)
            ja = (PGROUPS - 1) * K
            drain_gathers(0, gsem_a)
            sync_scatters(ja, 0)

        plsc.subcore_barrier()

        pltpu.sync_copy(acc.at[pl.ds(row0, ROWS_PER_TILE)],
                        out_hbm.at[c, pl.ds(row0, ROWS_PER_TILE)])

        @pl.when(s == NS - 1)
        def _():
            pltpu.sync_copy(acc.at[pl.ds(TAIL_ROW0, TAIL_ROWS)],
                            out_hbm.at[c, pl.ds(TAIL_ROW0, TAIL_ROWS)])

    return agg(h2, src, dst)


def _mlp_bn(p_ref, w1_ref, b1_ref, w2_ref, b2_ref, g_ref, bt_ref):
    z = jnp.concatenate([p_ref[0], p_ref[1]], axis=1)
    a = jnp.maximum(
        jnp.dot(z, w1_ref[...], preferred_element_type=jnp.float32)
        + b1_ref[...], 0.0)
    y = (jnp.dot(a, w2_ref[...], preferred_element_type=jnp.float32)
         + b2_ref[...])
    mean = jnp.mean(y, axis=0, keepdims=True)
    var = jnp.mean((y - mean) ** 2, axis=0, keepdims=True)
    hn = (y - mean) * lax.rsqrt(var + BN_EPS) * g_ref[...] + bt_ref[...]
    return jnp.maximum(hn, 0.0)


def _tc_layer(p, w1, b1, w2, b2, gamma, beta):
    def body(p_ref, w1_ref, b1_ref, w2_ref, b2_ref, g_ref, bt_ref, o_ref):
        h = _mlp_bn(p_ref, w1_ref, b1_ref, w2_ref, b2_ref, g_ref, bt_ref)
        o_ref[0] = h[:, :H]
        o_ref[1] = h[:, H:]

    return pl.pallas_call(
        body,
        out_shape=jax.ShapeDtypeStruct((NC, N, H), jnp.float32),
    )(p, w1, b1.reshape(1, D), w2, b2.reshape(1, D),
      gamma.reshape(1, D), beta.reshape(1, D))


def _tc_final(p, w1, b1, w2, b2, gamma, beta, batch2d, w_out, b_out):
    def body(p_ref, w1_ref, b1_ref, w2_ref, b2_ref, g_ref, bt_ref,
             batch_ref, wo_ref, bo_ref, o_ref):
        h = _mlp_bn(p_ref, w1_ref, b1_ref, w2_ref, b2_ref, g_ref, bt_ref)
        ids = batch_ref[...]
        onehot = (ids == lax.broadcasted_iota(jnp.int32, (1, G), 1)
                  ).astype(jnp.float32)
        counts = jnp.sum(onehot, axis=0, keepdims=True)
        inv = 1.0 / jnp.maximum(counts, 1.0)
        pooled = lax.dot_general(onehot * inv, h, (((0,), (0,)), ((), ())),
                                 preferred_element_type=jnp.float32)
        o_ref[...] = (jnp.dot(pooled, wo_ref[...],
                              preferred_element_type=jnp.float32)
                      + bo_ref[...])

    return pl.pallas_call(
        body,
        out_shape=jax.ShapeDtypeStruct((G, D), jnp.float32),
    )(p, w1, b1.reshape(1, D), w2, b2.reshape(1, D),
      gamma.reshape(1, D), beta.reshape(1, D),
      batch2d, w_out, b_out.reshape(1, D))


def kernel(x, edge_index, batch,
           w1_0, b1_0, w2_0, b2_0, gamma_0, beta_0,
           w1_1, b1_1, w2_1, b2_1, gamma_1, beta_1,
           w1_2, b1_2, w2_2, b2_2, gamma_2, beta_2,
           w_out, b_out):
    src = edge_index[0].reshape(NS, CHUNKS, CHUNK)
    dst = edge_index[1].reshape(NS, CHUNKS, CHUNK)
    batch2d = batch.reshape(N, 1)
    x2 = jnp.stack([x[:, :H], x[:, H:]])

    p = _sc_aggregate(x2, src, dst)
    h2 = _tc_layer(p, w1_0, b1_0, w2_0, b2_0, gamma_0, beta_0)
    p = _sc_aggregate(h2, src, dst)
    h2 = _tc_layer(p, w1_1, b1_1, w2_1, b2_1, gamma_1, beta_1)
    p = _sc_aggregate(h2, src, dst)
    return _tc_final(p, w1_2, b1_2, w2_2, b2_2, gamma_2, beta_2,
                     batch2d, w_out, b_out)
